# baseline (speedup 1.0000x reference)
import functools

import jax
import jax.numpy as jnp
from jax import lax
from jax.experimental import pallas as pl
from jax.experimental.pallas import tpu as pltpu
from jax.experimental.pallas import tpu_sc as plsc

KMAX = 16
KMIN = 16

NC = 2
NS = 16
NW = NC * NS
L = 16

TR = 64

SC_ROWS = 49152


def _sc_body(x_hbm, o_hbm, buf, outb):
    cid = lax.axis_index("c")
    sid = lax.axis_index("s")
    wid = sid * NC + cid
    rows_per_w = SC_ROWS // NW
    ntiles = rows_per_w // TR
    lane_iota = lax.iota(jnp.int32, L)

    def tile_body(t, carry):
        row0 = wid * rows_per_w + t * TR
        pltpu.sync_copy(x_hbm.at[pl.ds(row0, TR)], buf)

        def merge(a, b):
            at, ab = a
            bt, bb = b
            tm = jnp.maximum(at, lax.rev(bt, (0,)))
            bm = jnp.minimum(ab, lax.rev(bb, (0,)))
            ts, _ = plsc.sort_key_val(tm, tm)
            bs, _ = plsc.sort_key_val(bm, bm)
            return (ts, bs)

        def row_body(r, acc):
            slots = [None] * 7
            for j in range(36):
                c = buf[r, pl.ds(j * L, L)]
                cs, _ = plsc.sort_key_val(c, c)
                node = (cs, cs)
                k = 0
                while slots[k] is not None:
                    node = merge(slots[k], node)
                    slots[k] = None
                    k += 1
                slots[k] = node
            final = None
            for k in range(7):
                if slots[k] is not None:
                    final = slots[k] if final is None else merge(final, slots[k])
            top, bot = final
            val = (jnp.sum(top) + jnp.sum(bot)) * jnp.float32(0.5 / KMAX)
            lane = lax.rem(r, L)
            acc = jnp.where(lane_iota == lane, val, acc)

            @pl.when(lane == L - 1)
            def _():
                outb[pl.ds(r - (L - 1), L)] = acc

            return acc

        lax.fori_loop(0, TR, row_body, jnp.zeros((L,), jnp.float32))
        pltpu.sync_copy(outb, o_hbm.at[pl.ds(row0, TR)])
        return carry

    lax.fori_loop(0, ntiles, tile_body, jnp.int32(0))


def _sc_weldon(flat):
    mesh = plsc.VectorSubcoreMesh(core_axis_name="c", subcore_axis_name="s")
    return pl.kernel(
        _sc_body,
        out_type=jax.ShapeDtypeStruct((SC_ROWS,), jnp.float32),
        mesh=mesh,
        compiler_params=pltpu.CompilerParams(needs_layout_passes=False),
        scratch_types=[
            pltpu.VMEM((TR, 576), jnp.float32),
            pltpu.VMEM((TR,), jnp.float32),
        ],
    )(flat)


def _tc_body(x_ref, o_ref):
    x = x_ref[...]
    r, n = x.shape
    neg_inf = jnp.float32(-jnp.inf)
    pos_inf = jnp.float32(jnp.inf)

    xi = jax.lax.bitcast_convert_type(x, jnp.int32)
    idx = jax.lax.broadcasted_iota(jnp.int32, (r, n), 1)
    xu = jax.lax.bitcast_convert_type((xi & ~1023) | idx, jnp.float32)

    m_hi = jnp.max(xu, axis=1, keepdims=True)
    m_lo = jnp.min(xu, axis=1, keepdims=True)
    acc_hi = m_hi
    acc_lo = m_lo
    for _ in range(KMAX - 1):
        m_hi = jnp.max(jnp.where(xu < m_hi, xu, neg_inf), axis=1, keepdims=True)
        m_lo = jnp.min(jnp.where(xu > m_lo, xu, pos_inf), axis=1, keepdims=True)
        acc_hi = acc_hi + m_hi
        acc_lo = acc_lo + m_lo

    out = (acc_hi / KMAX + acc_lo / KMIN) * 0.5
    o_ref[0, 0] = out[:, 0]


def _tc_weldon(flat, row0):
    rows, n = flat.shape
    R = 512
    b0 = row0 // R
    g = rows // R - b0
    out = pl.pallas_call(
        _tc_body,
        grid=(g,),
        in_specs=[pl.BlockSpec((R, n), lambda i: (i + b0, 0))],
        out_specs=pl.BlockSpec((1, 1, R), lambda i: (i, 0, 0)),
        out_shape=jax.ShapeDtypeStruct((g, 1, R), jnp.float32),
    )(flat)
    return out.reshape(g * R)


def kernel(input):
    b, c, h, w = input.shape
    n = h * w
    rows = b * c
    flat = input.reshape(rows, n)
    parts = []
    if SC_ROWS > 0:
        parts.append(_sc_weldon(flat))
    if SC_ROWS < rows:
        parts.append(_tc_weldon(flat, SC_ROWS))
    out = parts[0] if len(parts) == 1 else jnp.concatenate(parts)
    return out.reshape(b, c)


# speedup vs baseline: 1.2193x; 1.2193x over previous
import functools

import jax
import jax.numpy as jnp
from jax import lax
from jax.experimental import pallas as pl
from jax.experimental.pallas import tpu as pltpu
from jax.experimental.pallas import tpu_sc as plsc

KMAX = 16
KMIN = 16

NC = 2
NS = 16
NW = NC * NS
L = 16

TR = 64

SC_ROWS = 32768


def _sc_body(x_hbm, o_hbm, buf0, buf1, outb, sem0, sem1):
    cid = lax.axis_index("c")
    sid = lax.axis_index("s")
    wid = sid * NC + cid
    rows_per_w = SC_ROWS // NW
    ntiles = rows_per_w // TR
    base = wid * rows_per_w
    lane_iota = lax.iota(jnp.int32, L)
    bufs = (buf0, buf1)
    sems = (sem0, sem1)

    def merge(a, b):
        at, ab = a
        bt, bb = b
        tm = jnp.maximum(at, lax.rev(bt, (0,)))
        bm = jnp.minimum(ab, lax.rev(bb, (0,)))
        ts, _ = plsc.sort_key_val(tm, tm)
        bs, _ = plsc.sort_key_val(bm, bm)
        return (ts, bs)

    def compute_tile(buf, row0):
        def row_body(r, acc):
            slots = [None] * 7
            for j in range(36):
                c = buf[r, pl.ds(j * L, L)]
                cs, _ = plsc.sort_key_val(c, c)
                node = (cs, cs)
                k = 0
                while slots[k] is not None:
                    node = merge(slots[k], node)
                    slots[k] = None
                    k += 1
                slots[k] = node
            final = None
            for k in range(7):
                if slots[k] is not None:
                    final = slots[k] if final is None else merge(final, slots[k])
            top, bot = final
            val = (jnp.sum(top) + jnp.sum(bot)) * jnp.float32(0.5 / KMAX)
            lane = lax.rem(r, L)
            acc = jnp.where(lane_iota == lane, val, acc)

            @pl.when(lane == L - 1)
            def _():
                outb[pl.ds(r - (L - 1), L)] = acc

            return acc

        lax.fori_loop(0, TR, row_body, jnp.zeros((L,), jnp.float32))
        pltpu.sync_copy(outb, o_hbm.at[pl.ds(row0, TR)])

    pltpu.async_copy(x_hbm.at[pl.ds(base, TR)], buf0, sem0)
    pltpu.async_copy(x_hbm.at[pl.ds(base + TR, TR)], buf1, sem1)

    def pair_body(p, carry):
        t0 = p * 2
        for b in range(2):
            t = t0 + b
            pltpu.make_async_copy(
                x_hbm.at[pl.ds(base + t * TR, TR)], bufs[b], sems[b]
            ).wait()
            compute_tile(bufs[b], base + t * TR)

            @pl.when(t + 2 < ntiles)
            def _():
                pltpu.async_copy(
                    x_hbm.at[pl.ds(base + (t + 2) * TR, TR)], bufs[b], sems[b]
                )
        return carry

    lax.fori_loop(0, ntiles // 2, pair_body, jnp.int32(0))


def _sc_weldon(flat):
    mesh = plsc.VectorSubcoreMesh(core_axis_name="c", subcore_axis_name="s")
    return pl.kernel(
        _sc_body,
        out_type=jax.ShapeDtypeStruct((SC_ROWS,), jnp.float32),
        mesh=mesh,
        compiler_params=pltpu.CompilerParams(needs_layout_passes=False),
        scratch_types=[
            pltpu.VMEM((TR, 576), jnp.float32),
            pltpu.VMEM((TR, 576), jnp.float32),
            pltpu.VMEM((TR,), jnp.float32),
            pltpu.SemaphoreType.DMA,
            pltpu.SemaphoreType.DMA,
        ],
    )(flat)


def _tc_body(x_ref, o_ref):
    x = x_ref[...]
    r, n = x.shape
    neg_inf = jnp.float32(-jnp.inf)
    pos_inf = jnp.float32(jnp.inf)

    xi = jax.lax.bitcast_convert_type(x, jnp.int32)
    idx = jax.lax.broadcasted_iota(jnp.int32, (r, n), 1)
    xu = jax.lax.bitcast_convert_type((xi & ~1023) | idx, jnp.float32)

    m_hi = jnp.max(xu, axis=1, keepdims=True)
    m_lo = jnp.min(xu, axis=1, keepdims=True)
    acc_hi = m_hi
    acc_lo = m_lo
    for _ in range(KMAX - 1):
        m_hi = jnp.max(jnp.where(xu < m_hi, xu, neg_inf), axis=1, keepdims=True)
        m_lo = jnp.min(jnp.where(xu > m_lo, xu, pos_inf), axis=1, keepdims=True)
        acc_hi = acc_hi + m_hi
        acc_lo = acc_lo + m_lo

    out = (acc_hi / KMAX + acc_lo / KMIN) * 0.5
    o_ref[0, 0] = out[:, 0]


def _tc_weldon(flat, row0):
    rows, n = flat.shape
    R = 512
    b0 = row0 // R
    g = rows // R - b0
    out = pl.pallas_call(
        _tc_body,
        grid=(g,),
        in_specs=[pl.BlockSpec((R, n), lambda i: (i + b0, 0))],
        out_specs=pl.BlockSpec((1, 1, R), lambda i: (i, 0, 0)),
        out_shape=jax.ShapeDtypeStruct((g, 1, R), jnp.float32),
    )(flat)
    return out.reshape(g * R)


def kernel(input):
    b, c, h, w = input.shape
    n = h * w
    rows = b * c
    flat = input.reshape(rows, n)
    parts = []
    if SC_ROWS > 0:
        parts.append(_sc_weldon(flat))
    if SC_ROWS < rows:
        parts.append(_tc_weldon(flat, SC_ROWS))
    out = parts[0] if len(parts) == 1 else jnp.concatenate(parts)
    return out.reshape(b, c)


# speedup vs baseline: 1.2281x; 1.0073x over previous
import functools

import jax
import jax.numpy as jnp
from jax import lax
from jax.experimental import pallas as pl
from jax.experimental.pallas import tpu as pltpu
from jax.experimental.pallas import tpu_sc as plsc

KMAX = 16
KMIN = 16

NC = 2
NS = 16
NW = NC * NS
L = 16

TR = 64

SC_ROWS = 36864


def _sc_body(x_hbm, o_hbm, buf0, buf1, outb, sem0, sem1):
    cid = lax.axis_index("c")
    sid = lax.axis_index("s")
    wid = sid * NC + cid
    rows_per_w = SC_ROWS // NW
    ntiles = rows_per_w // TR
    base = wid * rows_per_w
    lane_iota = lax.iota(jnp.int32, L)
    bufs = (buf0, buf1)
    sems = (sem0, sem1)

    def merge(a, b):
        at, ab = a
        bt, bb = b
        tm = jnp.maximum(at, lax.rev(bt, (0,)))
        bm = jnp.minimum(ab, lax.rev(bb, (0,)))
        ts, _ = plsc.sort_key_val(tm, tm)
        bs, _ = plsc.sort_key_val(bm, bm)
        return (ts, bs)

    def compute_tile(buf, row0):
        def row_body(r, acc):
            slots = [None] * 7
            for j in range(36):
                c = buf[r, pl.ds(j * L, L)]
                cs, _ = plsc.sort_key_val(c, c)
                node = (cs, cs)
                k = 0
                while slots[k] is not None:
                    node = merge(slots[k], node)
                    slots[k] = None
                    k += 1
                slots[k] = node
            final = None
            for k in range(7):
                if slots[k] is not None:
                    final = slots[k] if final is None else merge(final, slots[k])
            top, bot = final
            val = (jnp.sum(top) + jnp.sum(bot)) * jnp.float32(0.5 / KMAX)
            lane = lax.rem(r, L)
            acc = jnp.where(lane_iota == lane, val, acc)

            @pl.when(lane == L - 1)
            def _():
                outb[pl.ds(r - (L - 1), L)] = acc

            return acc

        lax.fori_loop(0, TR, row_body, jnp.zeros((L,), jnp.float32))
        pltpu.sync_copy(outb, o_hbm.at[pl.ds(row0, TR)])

    pltpu.async_copy(x_hbm.at[pl.ds(base, TR)], buf0, sem0)
    pltpu.async_copy(x_hbm.at[pl.ds(base + TR, TR)], buf1, sem1)

    def pair_body(p, carry):
        t0 = p * 2
        for b in range(2):
            t = t0 + b
            pltpu.make_async_copy(
                x_hbm.at[pl.ds(base + t * TR, TR)], bufs[b], sems[b]
            ).wait()
            compute_tile(bufs[b], base + t * TR)

            @pl.when(t + 2 < ntiles)
            def _():
                pltpu.async_copy(
                    x_hbm.at[pl.ds(base + (t + 2) * TR, TR)], bufs[b], sems[b]
                )
        return carry

    lax.fori_loop(0, ntiles // 2, pair_body, jnp.int32(0))


def _sc_weldon(flat):
    mesh = plsc.VectorSubcoreMesh(core_axis_name="c", subcore_axis_name="s")
    return pl.kernel(
        _sc_body,
        out_type=jax.ShapeDtypeStruct((SC_ROWS,), jnp.float32),
        mesh=mesh,
        compiler_params=pltpu.CompilerParams(needs_layout_passes=False),
        scratch_types=[
            pltpu.VMEM((TR, 576), jnp.float32),
            pltpu.VMEM((TR, 576), jnp.float32),
            pltpu.VMEM((TR,), jnp.float32),
            pltpu.SemaphoreType.DMA,
            pltpu.SemaphoreType.DMA,
        ],
    )(flat)


def _tc_body(x_ref, o_ref):
    x = x_ref[...]
    r, n = x.shape
    neg_inf = jnp.float32(-jnp.inf)
    pos_inf = jnp.float32(jnp.inf)

    xi = jax.lax.bitcast_convert_type(x, jnp.int32)
    idx = jax.lax.broadcasted_iota(jnp.int32, (r, n), 1)
    xu = jax.lax.bitcast_convert_type((xi & ~1023) | idx, jnp.float32)

    m_hi = jnp.max(xu, axis=1, keepdims=True)
    m_lo = jnp.min(xu, axis=1, keepdims=True)
    acc_hi = m_hi
    acc_lo = m_lo
    for _ in range(KMAX - 1):
        m_hi = jnp.max(jnp.where(xu < m_hi, xu, neg_inf), axis=1, keepdims=True)
        m_lo = jnp.min(jnp.where(xu > m_lo, xu, pos_inf), axis=1, keepdims=True)
        acc_hi = acc_hi + m_hi
        acc_lo = acc_lo + m_lo

    out = (acc_hi / KMAX + acc_lo / KMIN) * 0.5
    o_ref[0, 0] = out[:, 0]


def _tc_weldon(flat, row0):
    rows, n = flat.shape
    R = 512
    b0 = row0 // R
    g = rows // R - b0
    out = pl.pallas_call(
        _tc_body,
        grid=(g,),
        in_specs=[pl.BlockSpec((R, n), lambda i: (i + b0, 0))],
        out_specs=pl.BlockSpec((1, 1, R), lambda i: (i, 0, 0)),
        out_shape=jax.ShapeDtypeStruct((g, 1, R), jnp.float32),
    )(flat)
    return out.reshape(g * R)


def kernel(input):
    b, c, h, w = input.shape
    n = h * w
    rows = b * c
    flat = input.reshape(rows, n)
    parts = []
    if SC_ROWS > 0:
        parts.append(_sc_weldon(flat))
    if SC_ROWS < rows:
        parts.append(_tc_weldon(flat, SC_ROWS))
    out = parts[0] if len(parts) == 1 else jnp.concatenate(parts)
    return out.reshape(b, c)
